# baseline (speedup 1.0000x reference)
import jax
import jax.numpy as jnp
from jax import lax
from jax.experimental import pallas as pl
from jax.experimental.pallas import tpu as pltpu
from jax.experimental.pallas import tpu_sc as plsc

DIM = 1024
LANES = 16
NUM_CORES = 2
NUM_SUBCORES = 16
NUM_WORKERS = NUM_CORES * NUM_SUBCORES
CHUNK = 8
NB_O = 8
NB_PE = 4


def _make_kernel(n_rows):
    rows_per_worker = n_rows // NUM_WORKERS
    n_chunks = rows_per_worker // CHUNK
    assert n_chunks % NB_O == 0 and n_chunks >= 2 * NB_O
    mesh = plsc.VectorSubcoreMesh(core_axis_name="c", subcore_axis_name="s")

    @jax.jit
    def run(x, idx, pe):
        @pl.kernel(
            out_type=jax.ShapeDtypeStruct((n_rows, DIM), jnp.float32),
            mesh=mesh,
            scratch_types=[
                pltpu.VMEM((rows_per_worker,), jnp.int32),
                [pltpu.VMEM((CHUNK, DIM), jnp.float32)] * NB_PE,
                [pltpu.VMEM((CHUNK, DIM), jnp.float32)] * NB_O,
                [pltpu.SemaphoreType.DMA] * NB_PE,
                [pltpu.SemaphoreType.DMA] * NB_O,
            ],
        )
        def sc_kernel(x_hbm, idx_hbm, pe_hbm, out_hbm, idx_v, pe_v, o_v,
                      sem_in, sem_out):
            wid = lax.axis_index("s") * NUM_CORES + lax.axis_index("c")
            base = wid * rows_per_worker
            pltpu.sync_copy(idx_hbm.at[pl.ds(base, rows_per_worker)], idx_v)

            def start_in(g, bo, bp):
                pltpu.async_copy(
                    x_hbm.at[pl.ds(base + g * CHUNK, CHUNK)],
                    o_v[bo], sem_in[bp])
                pltpu.async_copy(
                    pe_hbm.at[idx_v.at[pl.ds(g * CHUNK, CHUNK)]],
                    pe_v[bp], sem_in[bp])

            def wait_in(bo, bp):
                pltpu.make_async_copy(
                    x_hbm.at[pl.ds(base, CHUNK)], pe_v[bp], sem_in[bp]).wait()
                pltpu.make_async_copy(
                    x_hbm.at[pl.ds(base, CHUNK)], o_v[bo], sem_in[bp]).wait()

            def wait_out(bo):
                pltpu.make_async_copy(
                    x_hbm.at[pl.ds(base, CHUNK)], o_v[bo], sem_out[bo]).wait()

            for g in range(NB_PE):
                start_in(g, g % NB_O, g % NB_PE)

            @pl.loop(0, n_chunks, step=NB_O)
            def _pipe(g0):
                for b in range(NB_O):
                    g = g0 + b
                    bp = b % NB_PE
                    wait_in(b, bp)

                    @pl.loop(0, CHUNK)
                    def _row(r):
                        for j in range(DIM // LANES):
                            sl = pl.ds(j * LANES, LANES)
                            plsc.addupdate(o_v[b].at[r, sl], pe_v[bp][r, sl])

                    pltpu.async_copy(
                        o_v[b], out_hbm.at[pl.ds(base + g * CHUNK, CHUNK)],
                        sem_out[b])

                    j_next = g + NB_PE
                    bo_next = (b + NB_PE) % NB_O

                    @pl.when(j_next < n_chunks)
                    def _():
                        @pl.when(j_next >= NB_O)
                        def _():
                            wait_out(bo_next)

                        start_in(j_next, bo_next, bp)

            for b in range(NB_O):
                wait_out(b)

        return sc_kernel(x, idx, pe)

    return run


def kernel(x, tss_indexes, pe):
    b, s, d = x.shape
    n_rows = b * s
    x_flat = x.reshape(n_rows, d)
    idx_flat = tss_indexes.reshape(n_rows).astype(jnp.int32)
    out = _make_kernel(n_rows)(x_flat, idx_flat, pe)
    return out.reshape(b, s, d)


# speedup vs baseline: 1.0133x; 1.0133x over previous
import jax
import jax.numpy as jnp
from jax import lax
from jax.experimental import pallas as pl
from jax.experimental.pallas import tpu as pltpu
from jax.experimental.pallas import tpu_sc as plsc

DIM = 1024
LANES = 16
NUM_CORES = 2
NUM_SUBCORES = 16
NUM_WORKERS = NUM_CORES * NUM_SUBCORES
CHUNK = 16
NB_O = 4
NB_PE = 2


def _make_kernel(n_rows):
    rows_per_worker = n_rows // NUM_WORKERS
    n_chunks = rows_per_worker // CHUNK
    assert n_chunks % NB_O == 0 and n_chunks >= 2 * NB_O
    mesh = plsc.VectorSubcoreMesh(core_axis_name="c", subcore_axis_name="s")

    @jax.jit
    def run(x, idx, pe):
        @pl.kernel(
            out_type=jax.ShapeDtypeStruct((n_rows, DIM), jnp.float32),
            mesh=mesh,
            scratch_types=[
                pltpu.VMEM((rows_per_worker,), jnp.int32),
                [pltpu.VMEM((CHUNK, DIM), jnp.float32)] * NB_PE,
                [pltpu.VMEM((CHUNK, DIM), jnp.float32)] * NB_O,
                [pltpu.SemaphoreType.DMA] * NB_PE,
                [pltpu.SemaphoreType.DMA] * NB_O,
            ],
        )
        def sc_kernel(x_hbm, idx_hbm, pe_hbm, out_hbm, idx_v, pe_v, o_v,
                      sem_in, sem_out):
            wid = lax.axis_index("s") * NUM_CORES + lax.axis_index("c")
            base = wid * rows_per_worker
            pltpu.sync_copy(idx_hbm.at[pl.ds(base, rows_per_worker)], idx_v)

            def start_in(g, bo, bp):
                pltpu.async_copy(
                    x_hbm.at[pl.ds(base + g * CHUNK, CHUNK)],
                    o_v[bo], sem_in[bp])
                pltpu.async_copy(
                    pe_hbm.at[idx_v.at[pl.ds(g * CHUNK, CHUNK)]],
                    pe_v[bp], sem_in[bp])

            def wait_in(bo, bp):
                pltpu.make_async_copy(
                    x_hbm.at[pl.ds(base, CHUNK)], pe_v[bp], sem_in[bp]).wait()
                pltpu.make_async_copy(
                    x_hbm.at[pl.ds(base, CHUNK)], o_v[bo], sem_in[bp]).wait()

            def wait_out(bo):
                pltpu.make_async_copy(
                    x_hbm.at[pl.ds(base, CHUNK)], o_v[bo], sem_out[bo]).wait()

            for g in range(NB_PE):
                start_in(g, g % NB_O, g % NB_PE)

            @pl.loop(0, n_chunks, step=NB_O)
            def _pipe(g0):
                for b in range(NB_O):
                    g = g0 + b
                    bp = b % NB_PE
                    wait_in(b, bp)

                    @pl.loop(0, CHUNK)
                    def _row(r):
                        for j in range(DIM // LANES):
                            sl = pl.ds(j * LANES, LANES)
                            plsc.addupdate(o_v[b].at[r, sl], pe_v[bp][r, sl])

                    pltpu.async_copy(
                        o_v[b], out_hbm.at[pl.ds(base + g * CHUNK, CHUNK)],
                        sem_out[b])

                    j_next = g + NB_PE
                    bo_next = (b + NB_PE) % NB_O

                    @pl.when(j_next < n_chunks)
                    def _():
                        @pl.when(j_next >= NB_O)
                        def _():
                            wait_out(bo_next)

                        start_in(j_next, bo_next, bp)

            for b in range(NB_O):
                wait_out(b)

        return sc_kernel(x, idx, pe)

    return run


def kernel(x, tss_indexes, pe):
    b, s, d = x.shape
    n_rows = b * s
    x_flat = x.reshape(n_rows, d)
    idx_flat = tss_indexes.reshape(n_rows).astype(jnp.int32)
    out = _make_kernel(n_rows)(x_flat, idx_flat, pe)
    return out.reshape(b, s, d)


# speedup vs baseline: 1.0564x; 1.0426x over previous
import jax
import jax.numpy as jnp
from jax import lax
from jax.experimental import pallas as pl
from jax.experimental.pallas import tpu as pltpu
from jax.experimental.pallas import tpu_sc as plsc

DIM = 1024
LANES = 16
NUM_CORES = 2
NUM_SUBCORES = 16
NUM_WORKERS = NUM_CORES * NUM_SUBCORES
CHUNK = 8
NBUF = 4


def _make_kernel(n_rows):
    rows_per_worker = n_rows // NUM_WORKERS
    n_chunks = rows_per_worker // CHUNK
    assert n_chunks % NBUF == 0 and n_chunks >= 2 * NBUF
    mesh = plsc.VectorSubcoreMesh(core_axis_name="c", subcore_axis_name="s")

    @jax.jit
    def run(x, idx, pe):
        @pl.kernel(
            out_type=jax.ShapeDtypeStruct((n_rows, DIM), jnp.float32),
            mesh=mesh,
            scratch_types=[
                pltpu.VMEM((rows_per_worker,), jnp.int32),
                [pltpu.VMEM((CHUNK, DIM), jnp.float32)] * NBUF,
                [pltpu.VMEM((CHUNK, DIM), jnp.float32)] * NBUF,
                [pltpu.SemaphoreType.DMA] * NBUF,
                [pltpu.SemaphoreType.DMA] * NBUF,
                pltpu.SemaphoreType.DMA,
            ],
        )
        def sc_kernel(x_hbm, idx_hbm, pe_hbm, out_hbm, idx_v, pe_v, o_v,
                      sem_in, sem_out, sem_idx):
            wid = lax.axis_index("s") * NUM_CORES + lax.axis_index("c")
            base = wid * rows_per_worker
            idx_cp = pltpu.async_copy(
                idx_hbm.at[pl.ds(base, rows_per_worker)], idx_v, sem_idx)

            def start_in(g, b):
                pltpu.async_copy(
                    x_hbm.at[pl.ds(base + g * CHUNK, CHUNK)],
                    o_v[b], sem_in[b])
                pltpu.async_copy(
                    pe_hbm.at[idx_v.at[pl.ds(g * CHUNK, CHUNK)]],
                    pe_v[b], sem_in[b])

            def wait_in(b):
                pltpu.make_async_copy(
                    x_hbm.at[pl.ds(base, CHUNK)], pe_v[b], sem_in[b]).wait()
                pltpu.make_async_copy(
                    x_hbm.at[pl.ds(base, CHUNK)], o_v[b], sem_in[b]).wait()

            def wait_out(b):
                pltpu.make_async_copy(
                    x_hbm.at[pl.ds(base, CHUNK)], o_v[b], sem_out[b]).wait()

            for b in range(NBUF):
                pltpu.async_copy(
                    x_hbm.at[pl.ds(base + b * CHUNK, CHUNK)],
                    o_v[b], sem_in[b])
            idx_cp.wait()
            for b in range(NBUF):
                pltpu.async_copy(
                    pe_hbm.at[idx_v.at[pl.ds(b * CHUNK, CHUNK)]],
                    pe_v[b], sem_in[b])

            @pl.loop(0, n_chunks, step=NBUF)
            def _pipe(g0):
                for b in range(NBUF):
                    g = g0 + b
                    q = (b + 2) % NBUF

                    @pl.when(jnp.logical_and(g >= 2, g + 2 < n_chunks))
                    def _():
                        wait_out(q)
                        start_in(g + 2, q)

                    wait_in(b)

                    @pl.loop(0, CHUNK)
                    def _row(r):
                        @pl.loop(0, DIM // LANES, unroll=16)
                        def _col(j):
                            sl = pl.ds(j * LANES, LANES)
                            plsc.addupdate(o_v[b].at[r, sl], pe_v[b][r, sl])

                    pltpu.async_copy(
                        o_v[b], out_hbm.at[pl.ds(base + g * CHUNK, CHUNK)],
                        sem_out[b])

            for b in range(NBUF):
                wait_out(b)

        return sc_kernel(x, idx, pe)

    return run


def kernel(x, tss_indexes, pe):
    b, s, d = x.shape
    n_rows = b * s
    x_flat = x.reshape(n_rows, d)
    idx_flat = tss_indexes.reshape(n_rows).astype(jnp.int32)
    out = _make_kernel(n_rows)(x_flat, idx_flat, pe)
    return out.reshape(b, s, d)
